# baseline (speedup 1.0000x reference)
import jax
import jax.numpy as jnp
from jax.experimental import pallas as pl
from jax.experimental.pallas import tpu as pltpu

H = 1024
E = 8
G = 4
I = 1408
AI = 128
TB = 512
BLK = 128
SCALE = 0.05
LB_COEF = 0.01


def _router_body(x_ref, w1x_ref, w1s_ref, b1_ref, w2_ref,
                 tv_ref, ti_ref, ps_ref, ms_ref, xb_ref):
    i = pl.program_id(0)
    x = x_ref[...]
    xb_ref[...] = x.astype(jnp.bfloat16)
    mean = jnp.mean(x, axis=1, keepdims=True)
    c = x - mean
    std = jnp.sqrt(jnp.sum(c * c, axis=1, keepdims=True) / (H - 1))
    mn = jnp.min(x, axis=1, keepdims=True)
    mx = jnp.max(x, axis=1, keepdims=True)
    l2 = jnp.sqrt(jnp.sum(x * x, axis=1, keepdims=True))
    sp = jnp.mean((jnp.abs(x) < 1e-6).astype(jnp.float32), axis=1, keepdims=True)
    stats = jnp.concatenate([mean, std, mn, mx, l2, sp], axis=1)

    hmid = jnp.dot(x, w1x_ref[...], preferred_element_type=jnp.float32)
    hmid = hmid + jnp.dot(stats, w1s_ref[...], preferred_element_type=jnp.float32)
    hmid = hmid + b1_ref[...]
    hmid = jax.nn.gelu(hmid)
    logits = jnp.dot(hmid, w2_ref[...], preferred_element_type=jnp.float32)

    m = jnp.max(logits, axis=1, keepdims=True)
    eexp = jnp.exp(logits - m)
    probs = eexp / jnp.sum(eexp, axis=1, keepdims=True)

    ie = jax.lax.broadcasted_iota(jnp.int32, (TB, E), 1)
    v1 = jnp.max(probs, axis=1, keepdims=True)
    i1 = jnp.min(jnp.where(probs == v1, ie, E), axis=1, keepdims=True)
    pm = jnp.where(ie == i1, -1.0, probs)
    v2 = jnp.max(pm, axis=1, keepdims=True)
    i2 = jnp.min(jnp.where(pm == v2, ie, E), axis=1, keepdims=True)
    tv_ref[...] = jnp.concatenate([v1, v2], axis=1)
    ti_ref[...] = jnp.concatenate([i1, i2], axis=1)

    oh = (((ie == i1) & (v1 > 0)).astype(jnp.float32)
          + (((ie == i2) & (v2 > 0))).astype(jnp.float32))

    @pl.when(i == 0)
    def _():
        ps_ref[...] = jnp.zeros_like(ps_ref)
        ms_ref[...] = jnp.zeros_like(ms_ref)

    ps_ref[...] += jnp.sum(probs, axis=0, keepdims=True)
    ms_ref[...] += jnp.sum(oh, axis=0, keepdims=True)


def _expert_body(te_ref, first_ref, slot_ref, nexte_ref, xg_ref, gate_ref,
                 up_hbm, dn_hbm, aup_ref, adn_ref, y_ref,
                 upf, dnf, upb, dnb, aupb, adnb, sem_up, sem_dn):
    i = pl.program_id(0)
    s = slot_ref[i]

    @pl.when(first_ref[i] == 1)
    def _():
        @pl.when(i == 0)
        def _():
            pltpu.make_async_copy(up_hbm.at[te_ref[0]], upf.at[0], sem_up.at[0]).start()
            pltpu.make_async_copy(dn_hbm.at[te_ref[0]], dnf.at[0], sem_dn.at[0]).start()
        pltpu.make_async_copy(up_hbm.at[te_ref[i]], upf.at[s], sem_up.at[s]).wait()
        pltpu.make_async_copy(dn_hbm.at[te_ref[i]], dnf.at[s], sem_dn.at[s]).wait()
        upb[...] = upf[s].astype(jnp.bfloat16)
        dnb[...] = dnf[s].astype(jnp.bfloat16)
        aupb[...] = aup_ref[0].astype(jnp.bfloat16)
        adnb[...] = adn_ref[0].astype(jnp.bfloat16)

        @pl.when(nexte_ref[i] >= 0)
        def _():
            ne = nexte_ref[i]
            pltpu.make_async_copy(up_hbm.at[ne], upf.at[1 - s], sem_up.at[1 - s]).start()
            pltpu.make_async_copy(dn_hbm.at[ne], dnf.at[1 - s], sem_dn.at[1 - s]).start()

    xb = xg_ref[...]
    h = jnp.dot(xb, upb[...], preferred_element_type=jnp.float32)
    g = h[:, :I]
    u = h[:, I:]
    act = g * jax.nn.sigmoid(g) * u
    y = jnp.dot(act.astype(jnp.bfloat16), dnb[...],
                preferred_element_type=jnp.float32)
    ha = jnp.dot(xb, aupb[...], preferred_element_type=jnp.float32)
    ga = ha[:, :AI]
    ua = ha[:, AI:]
    acta = ga * jax.nn.sigmoid(ga) * ua
    ya = jnp.dot(acta.astype(jnp.bfloat16), adnb[...],
                 preferred_element_type=jnp.float32)
    y_ref[...] = ((y + SCALE * ya) * gate_ref[...]).astype(jnp.bfloat16)


def kernel(x, r_w1, r_b1, r_w2, e_up, e_down, a_up, a_down):
    orig_shape = x.shape
    x2 = x.reshape(-1, H)
    T = x2.shape[0]
    P = 2 * T + E * BLK
    NT = P // BLK

    w1x = r_w1[:H]
    w1s = r_w1[H:]
    b1 = r_b1.reshape(1, -1)

    tv, ti, ps, ms, x2b = pl.pallas_call(
        _router_body,
        grid=(T // TB,),
        in_specs=[
            pl.BlockSpec((TB, H), lambda i: (i, 0)),
            pl.BlockSpec((H, H // 2), lambda i: (0, 0)),
            pl.BlockSpec((6, H // 2), lambda i: (0, 0)),
            pl.BlockSpec((1, H // 2), lambda i: (0, 0)),
            pl.BlockSpec((H // 2, E), lambda i: (0, 0)),
        ],
        out_specs=[
            pl.BlockSpec((TB, 2), lambda i: (i, 0)),
            pl.BlockSpec((TB, 2), lambda i: (i, 0)),
            pl.BlockSpec((1, E), lambda i: (0, 0)),
            pl.BlockSpec((1, E), lambda i: (0, 0)),
            pl.BlockSpec((TB, H), lambda i: (i, 0)),
        ],
        out_shape=[
            jax.ShapeDtypeStruct((T, 2), jnp.float32),
            jax.ShapeDtypeStruct((T, 2), jnp.int32),
            jax.ShapeDtypeStruct((1, E), jnp.float32),
            jax.ShapeDtypeStruct((1, E), jnp.float32),
            jax.ShapeDtypeStruct((T, H), jnp.bfloat16),
        ],
    )(x2, w1x, w1s, b1, r_w2)

    flat_e = ti.reshape(-1)
    flat_v = tv.reshape(-1)
    flat_t = (jnp.arange(2 * T, dtype=jnp.int32) // 2)
    ohf = (flat_e[:, None] == jnp.arange(E, dtype=jnp.int32)[None, :]).astype(jnp.float32)
    oh3 = ohf.reshape(2 * T // 128, 128, E)
    tril_strict = jnp.tril(jnp.ones((128, 128), jnp.float32), -1)
    loc = jnp.einsum('rk,nkg->nrg', tril_strict, oh3,
                     preferred_element_type=jnp.float32)
    bsum = oh3.sum(axis=1)
    bpre = jnp.cumsum(bsum, axis=0) - bsum
    rank3 = loc + bpre[:, None, :]
    myrank = jnp.sum(rank3.reshape(2 * T, E) * ohf, axis=1).astype(jnp.int32)
    counts = bsum.sum(axis=0).astype(jnp.int32)
    padded = ((counts + BLK - 1) // BLK) * BLK
    bounds = jnp.cumsum(padded)
    starts = bounds - padded
    dest = starts[flat_e] + myrank
    src_token = jnp.zeros((P,), jnp.int32).at[dest].set(flat_t)
    gate_sorted = jnp.zeros((P, 1), jnp.float32).at[dest, 0].set(flat_v)
    tile_start = jnp.arange(NT, dtype=jnp.int32) * BLK
    te = jnp.minimum(
        jnp.sum((tile_start[:, None] >= bounds[None, :]).astype(jnp.int32), axis=1),
        E - 1).astype(jnp.int32)
    first = jnp.concatenate(
        [jnp.ones((1,), jnp.int32), (te[1:] != te[:-1]).astype(jnp.int32)])
    seg_idx = jnp.cumsum(first) - 1
    slot = (seg_idx % 2).astype(jnp.int32)
    used = padded > 0
    ee = jnp.arange(E, dtype=jnp.int32)
    cand = jnp.where((ee[None, :] > ee[:, None]) & used[None, :], ee[None, :], E)
    nexte8 = jnp.min(cand, axis=1)
    nexte = jnp.where(nexte8[te] < E, nexte8[te], -1).astype(jnp.int32)

    xg = x2b[src_token]

    grid_spec = pltpu.PrefetchScalarGridSpec(
        num_scalar_prefetch=4,
        grid=(NT,),
        in_specs=[
            pl.BlockSpec((BLK, H), lambda i, *_: (i, 0)),
            pl.BlockSpec((BLK, 1), lambda i, *_: (i, 0)),
            pl.BlockSpec(memory_space=pltpu.MemorySpace.HBM),
            pl.BlockSpec(memory_space=pltpu.MemorySpace.HBM),
            pl.BlockSpec((1, H, 2 * AI), lambda i, te_r, *_: (te_r[i] // 2, 0, 0)),
            pl.BlockSpec((1, AI, H), lambda i, te_r, *_: (te_r[i] // 2, 0, 0)),
        ],
        out_specs=pl.BlockSpec((BLK, H), lambda i, *_: (i, 0)),
        scratch_shapes=[
            pltpu.VMEM((2, H, 2 * I), jnp.float32),
            pltpu.VMEM((2, I, H), jnp.float32),
            pltpu.VMEM((H, 2 * I), jnp.bfloat16),
            pltpu.VMEM((I, H), jnp.bfloat16),
            pltpu.VMEM((H, 2 * AI), jnp.bfloat16),
            pltpu.VMEM((AI, H), jnp.bfloat16),
            pltpu.SemaphoreType.DMA((2,)),
            pltpu.SemaphoreType.DMA((2,)),
        ],
    )
    y_p = pl.pallas_call(
        _expert_body,
        grid_spec=grid_spec,
        out_shape=jax.ShapeDtypeStruct((P, H), jnp.bfloat16),
    )(te, first, slot, nexte, xg, gate_sorted, e_up, e_down, a_up, a_down)

    d = dest.reshape(T, 2)
    yp2 = y_p[d]
    out = yp2[:, 0].astype(jnp.float32) + yp2[:, 1].astype(jnp.float32)
    aux = LB_COEF * E * jnp.sum((ms[0] / T) * (ps[0] / T))
    return out.reshape(orig_shape), aux


# speedup vs baseline: 1.1450x; 1.1450x over previous
import jax
import jax.numpy as jnp
from jax.experimental import pallas as pl
from jax.experimental.pallas import tpu as pltpu

H = 1024
E = 8
G = 4
I = 1408
AI = 128
TB = 512
BLK = 128
SCALE = 0.05
LB_COEF = 0.01


def _router_body(x_ref, w1x_ref, w1s_ref, b1_ref, w2_ref,
                 tv_ref, ti_ref, ps_ref, ms_ref, xb_ref):
    i = pl.program_id(0)
    x = x_ref[...]
    xb_ref[...] = x.astype(jnp.bfloat16)
    mean = jnp.mean(x, axis=1, keepdims=True)
    c = x - mean
    std = jnp.sqrt(jnp.sum(c * c, axis=1, keepdims=True) / (H - 1))
    mn = jnp.min(x, axis=1, keepdims=True)
    mx = jnp.max(x, axis=1, keepdims=True)
    l2 = jnp.sqrt(jnp.sum(x * x, axis=1, keepdims=True))
    sp = jnp.mean((jnp.abs(x) < 1e-6).astype(jnp.float32), axis=1, keepdims=True)
    stats = jnp.concatenate([mean, std, mn, mx, l2, sp], axis=1)

    hmid = jnp.dot(x, w1x_ref[...], preferred_element_type=jnp.float32)
    hmid = hmid + jnp.dot(stats, w1s_ref[...], preferred_element_type=jnp.float32)
    hmid = hmid + b1_ref[...]
    hmid = jax.nn.gelu(hmid)
    logits = jnp.dot(hmid, w2_ref[...], preferred_element_type=jnp.float32)

    m = jnp.max(logits, axis=1, keepdims=True)
    eexp = jnp.exp(logits - m)
    probs = eexp / jnp.sum(eexp, axis=1, keepdims=True)

    ie = jax.lax.broadcasted_iota(jnp.int32, (TB, E), 1)
    v1 = jnp.max(probs, axis=1, keepdims=True)
    i1 = jnp.min(jnp.where(probs == v1, ie, E), axis=1, keepdims=True)
    pm = jnp.where(ie == i1, -1.0, probs)
    v2 = jnp.max(pm, axis=1, keepdims=True)
    i2 = jnp.min(jnp.where(pm == v2, ie, E), axis=1, keepdims=True)
    tv_ref[...] = jnp.concatenate([v1, v2], axis=1)
    ti_ref[...] = jnp.concatenate([i1, i2], axis=1)

    oh = (((ie == i1) & (v1 > 0)).astype(jnp.float32)
          + (((ie == i2) & (v2 > 0))).astype(jnp.float32))

    @pl.when(i == 0)
    def _():
        ps_ref[...] = jnp.zeros_like(ps_ref)
        ms_ref[...] = jnp.zeros_like(ms_ref)

    ps_ref[...] += jnp.sum(probs, axis=0, keepdims=True)
    ms_ref[...] += jnp.sum(oh, axis=0, keepdims=True)


def _expert_body(te_ref, first_ref, slot_ref, nexte_ref, xg_ref, gate_ref,
                 up_hbm, dn_hbm, aup_ref, adn_ref, y_ref,
                 upf, dnf, upb, dnb, aupb, adnb, sem_up, sem_dn):
    i = pl.program_id(0)
    s = slot_ref[i]

    @pl.when(first_ref[i] == 1)
    def _():
        @pl.when(i == 0)
        def _():
            pltpu.make_async_copy(up_hbm.at[te_ref[0]], upf.at[0], sem_up.at[0]).start()
            pltpu.make_async_copy(dn_hbm.at[te_ref[0]], dnf.at[0], sem_dn.at[0]).start()
        pltpu.make_async_copy(up_hbm.at[te_ref[i]], upf.at[s], sem_up.at[s]).wait()
        pltpu.make_async_copy(dn_hbm.at[te_ref[i]], dnf.at[s], sem_dn.at[s]).wait()
        upb[...] = upf[s].astype(jnp.bfloat16)
        dnb[...] = dnf[s].astype(jnp.bfloat16)
        aupb[...] = aup_ref[0].astype(jnp.bfloat16)
        adnb[...] = adn_ref[0].astype(jnp.bfloat16)

        @pl.when(nexte_ref[i] >= 0)
        def _():
            ne = nexte_ref[i]
            pltpu.make_async_copy(up_hbm.at[ne], upf.at[1 - s], sem_up.at[1 - s]).start()
            pltpu.make_async_copy(dn_hbm.at[ne], dnf.at[1 - s], sem_dn.at[1 - s]).start()

    xb = xg_ref[...]
    h = jnp.dot(xb, upb[...], preferred_element_type=jnp.float32)
    g = h[:, :I]
    u = h[:, I:]
    act = g * jax.nn.sigmoid(g) * u
    y = jnp.dot(act.astype(jnp.bfloat16), dnb[...],
                preferred_element_type=jnp.float32)
    ha = jnp.dot(xb, aupb[...], preferred_element_type=jnp.float32)
    ga = ha[:, :AI]
    ua = ha[:, AI:]
    acta = ga * jax.nn.sigmoid(ga) * ua
    ya = jnp.dot(acta.astype(jnp.bfloat16), adnb[...],
                 preferred_element_type=jnp.float32)
    y_ref[...] = ((y + SCALE * ya) * gate_ref[...]).astype(jnp.bfloat16)


def kernel(x, r_w1, r_b1, r_w2, e_up, e_down, a_up, a_down):
    orig_shape = x.shape
    x2 = x.reshape(-1, H)
    T = x2.shape[0]
    P = 2 * T + E * BLK
    NT = P // BLK

    w1x = r_w1[:H]
    w1s = r_w1[H:]
    b1 = r_b1.reshape(1, -1)

    tv, ti, ps, ms, x2b = pl.pallas_call(
        _router_body,
        grid=(T // TB,),
        in_specs=[
            pl.BlockSpec((TB, H), lambda i: (i, 0)),
            pl.BlockSpec((H, H // 2), lambda i: (0, 0)),
            pl.BlockSpec((6, H // 2), lambda i: (0, 0)),
            pl.BlockSpec((1, H // 2), lambda i: (0, 0)),
            pl.BlockSpec((H // 2, E), lambda i: (0, 0)),
        ],
        out_specs=[
            pl.BlockSpec((TB, 2), lambda i: (i, 0)),
            pl.BlockSpec((TB, 2), lambda i: (i, 0)),
            pl.BlockSpec((1, E), lambda i: (0, 0)),
            pl.BlockSpec((1, E), lambda i: (0, 0)),
            pl.BlockSpec((TB, H), lambda i: (i, 0)),
        ],
        out_shape=[
            jax.ShapeDtypeStruct((T, 2), jnp.float32),
            jax.ShapeDtypeStruct((T, 2), jnp.int32),
            jax.ShapeDtypeStruct((1, E), jnp.float32),
            jax.ShapeDtypeStruct((1, E), jnp.float32),
            jax.ShapeDtypeStruct((T, H), jnp.bfloat16),
        ],
    )(x2, w1x, w1s, b1, r_w2)

    flat_e = ti.reshape(-1)
    flat_v = tv.reshape(-1)
    flat_t = (jnp.arange(2 * T, dtype=jnp.int32) // 2)
    ohf = (flat_e[:, None] == jnp.arange(E, dtype=jnp.int32)[None, :]).astype(jnp.float32)
    oh3 = ohf.reshape(2 * T // 128, 128, E)
    tril_strict = jnp.tril(jnp.ones((128, 128), jnp.float32), -1)
    loc = jnp.einsum('rk,nkg->nrg', tril_strict, oh3,
                     preferred_element_type=jnp.float32)
    bsum = oh3.sum(axis=1)
    bpre = jnp.cumsum(bsum, axis=0) - bsum
    rank3 = loc + bpre[:, None, :]
    myrank = jnp.sum(rank3.reshape(2 * T, E) * ohf, axis=1).astype(jnp.int32)
    counts = bsum.sum(axis=0).astype(jnp.int32)
    padded = ((counts + BLK - 1) // BLK) * BLK
    bounds = jnp.cumsum(padded)
    starts = bounds - padded
    dest = starts[flat_e] + myrank
    src_token = jnp.zeros((P,), jnp.int32).at[dest].set(flat_t)
    gate_sorted = jnp.zeros((P, 1), jnp.float32).at[dest, 0].set(flat_v)
    tile_start = jnp.arange(NT, dtype=jnp.int32) * BLK
    te = jnp.minimum(
        jnp.sum((tile_start[:, None] >= bounds[None, :]).astype(jnp.int32), axis=1),
        E - 1).astype(jnp.int32)
    first = jnp.concatenate(
        [jnp.ones((1,), jnp.int32), (te[1:] != te[:-1]).astype(jnp.int32)])
    seg_idx = jnp.cumsum(first) - 1
    slot = (seg_idx % 2).astype(jnp.int32)
    used = padded > 0
    ee = jnp.arange(E, dtype=jnp.int32)
    cand = jnp.where((ee[None, :] > ee[:, None]) & used[None, :], ee[None, :], E)
    nexte8 = jnp.min(cand, axis=1)
    nexte = jnp.where(nexte8[te] < E, nexte8[te], -1).astype(jnp.int32)

    xg = x2b[src_token]

    grid_spec = pltpu.PrefetchScalarGridSpec(
        num_scalar_prefetch=4,
        grid=(NT,),
        in_specs=[
            pl.BlockSpec((BLK, H), lambda i, *_: (i, 0)),
            pl.BlockSpec((BLK, 1), lambda i, *_: (i, 0)),
            pl.BlockSpec(memory_space=pltpu.MemorySpace.HBM),
            pl.BlockSpec(memory_space=pltpu.MemorySpace.HBM),
            pl.BlockSpec((1, H, 2 * AI), lambda i, te_r, *_: (te_r[i] // 2, 0, 0)),
            pl.BlockSpec((1, AI, H), lambda i, te_r, *_: (te_r[i] // 2, 0, 0)),
        ],
        out_specs=pl.BlockSpec((BLK, H), lambda i, *_: (i, 0)),
        scratch_shapes=[
            pltpu.VMEM((2, H, 2 * I), jnp.float32),
            pltpu.VMEM((2, I, H), jnp.float32),
            pltpu.VMEM((H, 2 * I), jnp.bfloat16),
            pltpu.VMEM((I, H), jnp.bfloat16),
            pltpu.VMEM((H, 2 * AI), jnp.bfloat16),
            pltpu.VMEM((AI, H), jnp.bfloat16),
            pltpu.SemaphoreType.DMA((2,)),
            pltpu.SemaphoreType.DMA((2,)),
        ],
    )
    y_p = pl.pallas_call(
        _expert_body,
        grid_spec=grid_spec,
        out_shape=jax.ShapeDtypeStruct((P, H), jnp.bfloat16),
    )(te, first, slot, nexte, xg, gate_sorted, e_up, e_down, a_up, a_down)

    d = dest.reshape(T, 2)
    out = (y_p[d[:, 0]].astype(jnp.float32) + y_p[d[:, 1]].astype(jnp.float32))
    aux = LB_COEF * E * jnp.sum((ms[0] / T) * (ps[0] / T))
    return out.reshape(orig_shape), aux
